# Initial kernel scaffold; baseline (speedup 1.0000x reference)
import jax
import jax.numpy as jnp
from jax.experimental import pallas as pl


def kernel(input_seqs, table):
    raise NotImplementedError("write your pallas kernel here")


# speedup vs baseline: 1.4995x; 1.4995x over previous
import functools

import jax
import jax.numpy as jnp
from jax import lax
from jax.experimental import pallas as pl
from jax.experimental.pallas import tpu as pltpu
from jax.experimental.pallas import tpu_sc as plsc

B, L, D = 4096, 200, 32
N = B * L
NC, NS = 2, 16
NW = NC * NS
PER_W = N // NW
CHUNK = 128
NCHUNK = PER_W // CHUNK
GROUP = 20
NGROUP = NCHUNK // GROUP
GROUP_ROWS = GROUP * CHUNK

_mesh = plsc.VectorSubcoreMesh(core_axis_name="c", subcore_axis_name="s")


@functools.partial(
    pl.kernel,
    mesh=_mesh,
    out_type=jax.ShapeDtypeStruct((N, D), jnp.float32),
    scratch_types=[
        pltpu.VMEM((NCHUNK, CHUNK), jnp.int32),
        pltpu.VMEM((GROUP_ROWS, D), jnp.float32),
        pltpu.SemaphoreType.DMA,
    ],
    compiler_params=pltpu.CompilerParams(use_tc_tiling_on_sc=False),
)
def _gather_kernel(table_hbm, idx_hbm, out_hbm, idx_v, rows_v, sem):
    wid = lax.axis_index("s") * NC + lax.axis_index("c")
    pltpu.sync_copy(idx_hbm.at[pl.ds(wid * NCHUNK, NCHUNK)], idx_v)

    def body(g, carry):
        handles = [
            pltpu.async_copy(
                table_hbm.at[idx_v.at[g * GROUP + j]],
                rows_v.at[pl.ds(j * CHUNK, CHUNK)],
                sem,
            )
            for j in range(GROUP)
        ]
        for h in handles:
            h.wait()
        pltpu.sync_copy(
            rows_v,
            out_hbm.at[pl.ds(wid * PER_W + g * GROUP_ROWS, GROUP_ROWS)],
        )
        return carry

    lax.fori_loop(0, NGROUP, body, 0)


def kernel(input_seqs, table):
    idx = input_seqs.reshape(N // CHUNK, CHUNK)
    out = _gather_kernel(table, idx)
    return out.reshape(B, L, D)
